# Initial kernel scaffold; baseline (speedup 1.0000x reference)
import jax
import jax.numpy as jnp
from jax.experimental import pallas as pl


def kernel(model_out, gold):
    raise NotImplementedError("write your pallas kernel here")


# speedup vs baseline: 9.4599x; 9.4599x over previous
import jax
import jax.numpy as jnp
from jax.experimental import pallas as pl
from jax.experimental.pallas import tpu as pltpu

_LS = 0.1
_V = 32000
_PAD = 0
_N = 2048
_BLOCK = 128
_NB = _N // _BLOCK
_SMOOTH = _LS / (_V - 2)
_CONF = 1.0 - _LS


def _ls_kernel(x_ref, g_ref, out_ref, acc_ref, cnt_ref):
    i = pl.program_id(0)
    x = x_ref[...]
    g = g_ref[0, 0, :]
    m = jnp.max(x, axis=1)
    z = jnp.sum(jnp.exp(x - m[:, None]), axis=1)
    L = m + jnp.log(z)
    T = jnp.sum(x, axis=1)
    col = jax.lax.broadcasted_iota(jnp.int32, (_BLOCK, _V), 1)
    xg = jnp.sum(jnp.where(col == g[:, None], x, 0.0), axis=1)
    x0 = x[:, 0]
    S = T - _V * L
    pg = xg - L
    p0 = x0 - L
    c = _SMOOTH * (S - p0 - pg) + _CONF * pg
    valid = g != _PAD
    part = jnp.sum(jnp.where(valid, -c, 0.0))
    cnt = jnp.sum(valid.astype(jnp.float32))

    @pl.when(i == 0)
    def _():
        acc_ref[0, 0] = 0.0
        cnt_ref[0, 0] = 0.0

    acc_ref[0, 0] += part
    cnt_ref[0, 0] += cnt

    @pl.when(i == _NB - 1)
    def _():
        out_ref[0, 0] = acc_ref[0, 0] / cnt_ref[0, 0]


def kernel(model_out, gold):
    out = pl.pallas_call(
        _ls_kernel,
        grid=(_NB,),
        in_specs=[
            pl.BlockSpec((_BLOCK, _V), lambda i: (i, 0)),
            pl.BlockSpec((1, 1, _BLOCK), lambda i: (i, 0, 0)),
        ],
        out_specs=pl.BlockSpec(memory_space=pltpu.SMEM),
        out_shape=jax.ShapeDtypeStruct((1, 1), jnp.float32),
        scratch_shapes=[
            pltpu.SMEM((1, 1), jnp.float32),
            pltpu.SMEM((1, 1), jnp.float32),
        ],
    )(model_out, gold.reshape(_NB, 1, _BLOCK))
    return out[0, 0]
